# Initial kernel scaffold; baseline (speedup 1.0000x reference)
import jax
import jax.numpy as jnp
from jax.experimental import pallas as pl


def kernel(z, edges):
    raise NotImplementedError("write your pallas kernel here")


# speedup vs baseline: 4.4038x; 4.4038x over previous
import functools

import jax
import jax.numpy as jnp
from jax import lax
from jax.experimental import pallas as pl
from jax.experimental.pallas import tpu as pltpu
from jax.experimental.pallas import tpu_sc as plsc

NC = 2
NS = 16
NW = NC * NS
L = 16

D = 128
E = 320000
EPW = E // NW
C = 400
UNROLL = 8


@functools.lru_cache(maxsize=None)
def _build():
  mesh = plsc.VectorSubcoreMesh(core_axis_name="c", subcore_axis_name="s")

  @functools.partial(
      pl.kernel,
      mesh=mesh,
      compiler_params=pltpu.CompilerParams(needs_layout_passes=False),
      out_type=jax.ShapeDtypeStruct((E,), jnp.float32),
      scratch_types=[
          pltpu.VMEM((C,), jnp.int32),
          pltpu.VMEM((C,), jnp.int32),
          pltpu.VMEM((C, D), jnp.float32),
          pltpu.VMEM((C, D), jnp.float32),
          pltpu.VMEM((C,), jnp.float32),
          pltpu.SemaphoreType.DMA,
      ],
  )
  def sc_kernel(z_hbm, a_hbm, b_hbm, out_hbm, aidx, bidx, u_v, v_v, o_v, sem):
    wid = lax.axis_index("s") * NC + lax.axis_index("c")
    wbase = wid * EPW
    lane = lax.iota(jnp.int32, L)

    def chunk_body(c, carry):
      base = wbase + c * C
      pltpu.sync_copy(a_hbm.at[pl.ds(base, C)], aidx)
      pltpu.sync_copy(b_hbm.at[pl.ds(base, C)], bidx)
      cp_u = pltpu.async_copy(z_hbm.at[aidx], u_v, sem)
      cp_v = pltpu.async_copy(z_hbm.at[bidx], v_v, sem)
      cp_u.wait()
      cp_v.wait()

      m15 = lane == 15

      def edge_body(eb, carry2):
        for de in range(UNROLL):
          e = eb * UNROLL + de
          acc = jnp.zeros((L,), jnp.float32)
          for k in range(D // L):
            ua = u_v[e, pl.ds(k * L, L)]
            vb = v_v[e, pl.ds(k * L, L)]
            acc = acc + ua * vb
          s = plsc.cumsum(acc)
          plsc.store_scatter(o_v, [jnp.full((L,), 0, jnp.int32) + e], s,
                             mask=m15)
        return carry2

      lax.fori_loop(0, C // UNROLL, edge_body, 0)
      pltpu.sync_copy(o_v, out_hbm.at[pl.ds(base, C)])
      return carry

    lax.fori_loop(0, EPW // C, chunk_body, 0)

  return sc_kernel


def kernel(z, edges):
  a = edges[:, 0]
  b = edges[:, 1]
  return _build()(z, a, b)


# speedup vs baseline: 6.1910x; 1.4058x over previous
import functools

import jax
import jax.numpy as jnp
from jax import lax
from jax.experimental import pallas as pl
from jax.experimental.pallas import tpu as pltpu
from jax.experimental.pallas import tpu_sc as plsc

NC = 2
NS = 16
NW = NC * NS
L = 16

D = 128
E = 320000
EPW = E // NW
C = 200
NCHUNKS = EPW // C


@functools.lru_cache(maxsize=None)
def _build():
  mesh = plsc.VectorSubcoreMesh(core_axis_name="c", subcore_axis_name="s")

  @functools.partial(
      pl.kernel,
      mesh=mesh,
      compiler_params=pltpu.CompilerParams(needs_layout_passes=False),
      out_type=jax.ShapeDtypeStruct((E,), jnp.float32),
      scratch_types=[
          pltpu.VMEM((C,), jnp.int32),
          pltpu.VMEM((C,), jnp.int32),
          pltpu.VMEM((C,), jnp.int32),
          pltpu.VMEM((C,), jnp.int32),
          pltpu.VMEM((C, D), jnp.float32),
          pltpu.VMEM((C, D), jnp.float32),
          pltpu.VMEM((C, D), jnp.float32),
          pltpu.VMEM((C, D), jnp.float32),
          pltpu.VMEM((C,), jnp.float32),
          pltpu.VMEM((L * L,), jnp.float32),
          pltpu.SemaphoreType.DMA,
          pltpu.SemaphoreType.DMA,
      ],
  )
  def sc_kernel(z_hbm, a_hbm, b_hbm, out_hbm,
                aidx0, bidx0, aidx1, bidx1,
                u0, v0, u1, v1, o_v, accb, sem0, sem1):
    wid = lax.axis_index("s") * NC + lax.axis_index("c")
    wbase = wid * EPW
    lane = lax.iota(jnp.int32, L)
    lane16 = lane * L

    slots = ((aidx0, bidx0, u0, v0, sem0), (aidx1, bidx1, u1, v1, sem1))

    def issue(t, s):
      aidx, bidx, u_v, v_v, sem = slots[s]
      base = wbase + t * C
      pltpu.sync_copy(a_hbm.at[pl.ds(base, C)], aidx)
      pltpu.sync_copy(b_hbm.at[pl.ds(base, C)], bidx)
      pltpu.make_async_copy(z_hbm.at[aidx], u_v, sem).start()
      pltpu.make_async_copy(z_hbm.at[bidx], v_v, sem).start()

    def wait(s):
      aidx, bidx, u_v, v_v, sem = slots[s]
      pltpu.make_async_copy(z_hbm.at[aidx], u_v, sem).wait()
      pltpu.make_async_copy(z_hbm.at[bidx], v_v, sem).wait()

    def compute(s):
      _, _, u_v, v_v, _ = slots[s]

      def group(g, carry2):
        gbase = g * L
        for el in range(L):
          e = gbase + el
          p = []
          for k in range(D // L):
            ua = u_v[e, pl.ds(k * L, L)]
            vb = v_v[e, pl.ds(k * L, L)]
            p.append(ua * vb)
          q = [p[0] + p[1], p[2] + p[3], p[4] + p[5], p[6] + p[7]]
          acc = (q[0] + q[1]) + (q[2] + q[3])
          accb[pl.ds(el * L, L)] = acc
        red = plsc.load_gather(accb, [lane16])
        for j in range(1, L):
          red = red + plsc.load_gather(accb, [lane16 + j])
        o_v[pl.ds(gbase, L)] = red
        return carry2

      lax.fori_loop(0, C // L, group, 0)

    issue(0, 0)

    def outer(g, carry):
      for b in (0, 1):
        t = g * 2 + b

        @pl.when(t + 1 < NCHUNKS)
        def _():
          issue(t + 1, 1 - b)

        wait(b)
        compute(b)
        pltpu.sync_copy(o_v, out_hbm.at[pl.ds(wbase + t * C, C)])
      return carry

    lax.fori_loop(0, NCHUNKS // 2, outer, 0)

  return sc_kernel


def kernel(z, edges):
  a = edges[:, 0]
  b = edges[:, 1]
  return _build()(z, a, b)
